# baseline (speedup 1.0000x reference)
import functools

import jax
import jax.numpy as jnp
from jax import lax
from jax.experimental import pallas as pl
from jax.experimental.pallas import tpu as pltpu
from jax.experimental.pallas import tpu_sc as plsc

CONTACT_IDX = (8, 10, 20, 23, 31)
INSIDE_IDX = 5
DIST_SQ_THRESH = 4.0

_LBL_MASK_U = 0
for _ci in (INSIDE_IDX,) + CONTACT_IDX:
    _LBL_MASK_U |= 1 << _ci
_LBL_MASK = _LBL_MASK_U - (1 << 32) if _LBL_MASK_U >= (1 << 31) else _LBL_MASK_U

NC = 2
NS = 16
NW = NC * NS

_SC_PARAMS = pltpu.CompilerParams(
    needs_layout_passes=False, use_tc_tiling_on_sc=False)

_TW = 16
_SN = 80

_CHUNK = 2000
_GB = 80
_NBMAX = _CHUNK // _GB
_WB = 3
_WAVE = _WB * _GB
_NWAVE = (_NBMAX + _WB - 1) // _WB
_SCAN_UNROLL = 5


def _mesh():
    return plsc.VectorSubcoreMesh(core_axis_name="c", subcore_axis_name="s")


def _full(c):
    return jnp.full((16,), c, jnp.int32)


def _make_fused(k_edges, n_nodes):
    pw = k_edges // NW
    nchunk = pw // _CHUNK
    assert pw % _CHUNK == 0 and pw % 8 == 0
    assert nchunk >= 3 and nchunk % 2 == 1

    nwin = -(-n_nodes // (16 * NS)) * 16
    nsub = -(-nwin // _SN)
    assert nwin % 16 == 0 and nwin % 8 == 0 and (n_nodes - nwin) % 8 == 0

    @functools.partial(
        pl.kernel,
        mesh=_mesh(),
        compiler_params=_SC_PARAMS,
        out_type=jax.ShapeDtypeStruct((k_edges,), jnp.int32),
        scratch_types=[
            pltpu.VMEM((_CHUNK,), jnp.int32),
            pltpu.VMEM((_CHUNK,), jnp.int32),
            pltpu.VMEM((_CHUNK,), jnp.int32),
            pltpu.VMEM((_CHUNK,), jnp.int32),
            pltpu.VMEM((_CHUNK,), jnp.int32),
            pltpu.VMEM((_CHUNK,), jnp.int32),
            pltpu.VMEM((_CHUNK + 16,), jnp.int32),
            pltpu.VMEM((_NBMAX, _GB), jnp.int32),
            pltpu.VMEM((_NBMAX, _GB), jnp.int32),
            pltpu.VMEM((_WAVE, _TW), jnp.float32),
            pltpu.VMEM((_WAVE, _TW), jnp.float32),
            pltpu.VMEM((24, _SN), jnp.float32),
            pltpu.VMEM((24, _SN), jnp.float32),
            pltpu.VMEM((_SN, _TW), jnp.float32),
            pltpu.VMEM_SHARED((n_nodes, _TW), jnp.float32),
            pltpu.SemaphoreType.DMA,
            pltpu.SemaphoreType.DMA,
            pltpu.SemaphoreType.DMA,
            pltpu.SemaphoreType.DMA,
        ],
    )
    def fused(ct_hbm, pidx_hbm, oidx_hbm, lbl_hbm, out_hbm,
              pidx_v, oidx_v, lbl_a, lbl_b, out_a, out_b, cidx_v,
              cpi_v, coi_v, prow_v, orow_v, cin_a, cin_b, sout_v, stats_sh,
              sem_in, sem_lbl, sem_out, sem_g):
        wid = lax.axis_index("s") * NC + lax.axis_index("c")
        base = wid * pw
        iota16 = lax.iota(jnp.int32, 16)
        ones16 = jnp.ones((16,), jnp.int32)
        zeros16 = jnp.zeros((16,), jnp.int32)
        bmask = jnp.full((16,), _LBL_MASK, jnp.int32)

        nwin_base = jnp.minimum(lax.axis_index("s") * nwin, n_nodes - nwin)

        def sub_base(c):
            return nwin_base + jnp.minimum(c * _SN, nwin - _SN)

        def fire_stats(c, buf):
            pltpu.async_copy(
                ct_hbm.at[:, pl.ds(sub_base(c), _SN)], buf, sem_in)

        def stats_sub(c, cin, nxt, fire_next):
            pltpu.make_async_copy(
                ct_hbm.at[:, pl.ds(0, _SN)], cin, sem_in).wait()
            if fire_next:
                fire_stats(c + 1, nxt)
            for g in range(_SN // 16):
                sl = pl.ds(g * 16, 16)
                rows = g * 16 + iota16
                for k in range(3):
                    vs = [cin[k * 8 + cc, sl] for cc in range(8)]
                    acc = vs[0]
                    mn = vs[0]
                    mx = vs[0]
                    for v in vs[1:]:
                        acc = acc + v
                        mn = jnp.minimum(mn, v)
                        mx = jnp.maximum(mx, v)
                    plsc.store_scatter(sout_v, [rows, _full(k)], acc * 0.125)
                    plsc.store_scatter(sout_v, [rows, _full(3 + k)], mn)
                    plsc.store_scatter(sout_v, [rows, _full(6 + k)], mx)
            pltpu.sync_copy(sout_v, stats_sh.at[pl.ds(sub_base(c), _SN)])

        fire_stats(0, cin_a)

        def stats_pair(c2, carry):
            stats_sub(2 * c2, cin_a, cin_b, True)
            stats_sub(2 * c2 + 1, cin_b, cin_a, True)
            return carry

        if nsub % 2 == 1:
            lax.fori_loop(0, nsub // 2, stats_pair, 0)
            stats_sub(nsub - 1, cin_a, cin_b, False)
        else:
            lax.fori_loop(0, nsub // 2 - 1, stats_pair, 0)
            stats_sub(nsub - 2, cin_a, cin_b, True)
            stats_sub(nsub - 1, cin_b, cin_a, False)

        plsc.subcore_barrier()

        def init_body(i, carry):
            pos = i * 16 + iota16
            plsc.store_scatter(cpi_v, [pos // _GB, pos % _GB], zeros16)
            plsc.store_scatter(coi_v, [pos // _GB, pos % _GB], zeros16)
            return carry

        lax.fori_loop(0, _CHUNK // 16, init_body, 0)

        pltpu.async_copy(lbl_hbm.at[pl.ds(base, _CHUNK)], lbl_a, sem_lbl)

        def process(k, lbl_v, out_v, nxt_v, fire_next):
            cbase = base + k * _CHUNK
            cp_p = pltpu.async_copy(
                pidx_hbm.at[pl.ds(cbase, _CHUNK)], pidx_v, sem_in)
            cp_o = pltpu.async_copy(
                oidx_hbm.at[pl.ds(cbase, _CHUNK)], oidx_v, sem_in)
            pltpu.make_async_copy(
                lbl_hbm.at[pl.ds(base, _CHUNK)], lbl_v, sem_lbl).wait()
            if fire_next:
                pltpu.async_copy(
                    lbl_hbm.at[pl.ds(cbase + _CHUNK, _CHUNK)], nxt_v, sem_lbl)
            @pl.when(k >= 2)
            def _wait_out():
                pltpu.make_async_copy(
                    out_v, out_hbm.at[pl.ds(base, _CHUNK)], sem_out).wait()

            def scan_body(i, cnt):
                for u in range(_SCAN_UNROLL):
                    g = i * _SCAN_UNROLL + u
                    sl = pl.ds(g * 16, 16)
                    lbl = lbl_v[sl]
                    bit = lax.shift_right_logical(bmask, jnp.minimum(lbl, 31))
                    m = ((bit & 1) != 0) & (lbl <= 31)
                    out_v[sl] = ones16
                    plsc.store_compressed(
                        cidx_v.at[pl.ds(cnt, 16)], g * 16 + iota16, mask=m)
                    cnt = cnt + jnp.sum(m.astype(jnp.int32))
                return cnt

            cnt = lax.fori_loop(0, _CHUNK // (16 * _SCAN_UNROLL), scan_body, 0)
            ngrp = (cnt + 15) // 16
            cp_p.wait()
            cp_o.wait()

            def b1_body(g, carry2):
                sl = pl.ds(g * 16, 16)
                pos = g * 16 + iota16
                valid = pos < cnt
                eid = jnp.where(valid, cidx_v[sl], 0)
                plsc.store_scatter(cpi_v, [pos // _GB, pos % _GB],
                                   plsc.load_gather(pidx_v, [eid]))
                plsc.store_scatter(coi_v, [pos // _GB, pos % _GB],
                                   plsc.load_gather(oidx_v, [eid]))
                return carry2

            lax.fori_loop(0, ngrp, b1_body, 0)

            for w in range(_NWAVE):
                wb0 = w * _WAVE
                nb = min(_WB, _NBMAX - w * _WB)
                for b in range(nb):
                    @pl.when(wb0 + b * _GB < cnt)
                    def _fire(b=b, w=w):
                        sl = pl.ds(b * _GB, _GB)
                        pltpu.async_copy(
                            stats_sh.at[cpi_v.at[w * _WB + b]], prow_v.at[sl],
                            sem_g)
                        pltpu.async_copy(
                            stats_sh.at[coi_v.at[w * _WB + b]], orow_v.at[sl],
                            sem_g)
                for b in range(nb):
                    @pl.when(wb0 + b * _GB < cnt)
                    def _drain(b=b, w=w):
                        sl = pl.ds(b * _GB, _GB)
                        pltpu.make_async_copy(
                            stats_sh.at[cpi_v.at[w * _WB + b]], prow_v.at[sl],
                            sem_g).wait()
                        pltpu.make_async_copy(
                            stats_sh.at[coi_v.at[w * _WB + b]], orow_v.at[sl],
                            sem_g).wait()

                def b3_body(g, carry2, wb0=wb0):
                    rows = g * 16 + iota16
                    pos = wb0 + g * 16 + iota16
                    valid = pos < cnt
                    eid = jnp.where(
                        valid, cidx_v[pl.ds(wb0 + g * 16, 16)], 0)
                    lbl = plsc.load_gather(lbl_v, [eid])

                    def pcol(c):
                        return plsc.load_gather(prow_v, [rows, _full(c)])

                    def ocol(c):
                        return plsc.load_gather(orow_v, [rows, _full(c)])

                    ox, oy, oz = ocol(0), ocol(1), ocol(2)
                    dx = pcol(0) - ox
                    dy = pcol(1) - oy
                    dz = pcol(2) - oz
                    d2 = dx * dx + dy * dy + dz * dz
                    inb = ((ox >= pcol(3)) & (oy >= pcol(4)) & (oz >= pcol(5))
                           & (ox <= pcol(6)) & (oy <= pcol(7))
                           & (oz <= pcol(8)))
                    veto_m = jnp.where(lbl == INSIDE_IDX, ~inb,
                                       d2 > DIST_SQ_THRESH)
                    plsc.store_scatter(out_v, [eid], zeros16,
                                       mask=veto_m & valid)
                    return carry2

                ngrpw = jnp.clip(ngrp - wb0 // 16, 0, _WAVE // 16)
                lax.fori_loop(0, ngrpw, b3_body, 0)

            pltpu.async_copy(out_v, out_hbm.at[pl.ds(cbase, _CHUNK)], sem_out)

        def pair_body(k2, carry):
            process(2 * k2, lbl_a, out_a, lbl_b, True)
            process(2 * k2 + 1, lbl_b, out_b, lbl_a, True)
            return carry

        lax.fori_loop(0, nchunk // 2, pair_body, 0)
        process(nchunk - 1, lbl_a, out_a, lbl_b, False)
        pltpu.make_async_copy(
            out_a, out_hbm.at[pl.ds(base, _CHUNK)], sem_out).wait()
        pltpu.make_async_copy(
            out_b, out_hbm.at[pl.ds(base, _CHUNK)], sem_out).wait()

    return fused


def kernel(corners, person_idx, object_idx, pred_labels):
    n = corners.shape[0]
    k = person_idx.shape[0]
    ct = corners.transpose(2, 1, 0).reshape(24, n)
    keep32 = _make_fused(k, n)(ct,
                               person_idx.astype(jnp.int32),
                               object_idx.astype(jnp.int32),
                               pred_labels.astype(jnp.int32))
    return keep32.astype(jnp.bool_)


# speedup vs baseline: 1.2529x; 1.2529x over previous
import functools

import jax
import jax.numpy as jnp
from jax import lax
from jax.experimental import pallas as pl
from jax.experimental.pallas import tpu as pltpu
from jax.experimental.pallas import tpu_sc as plsc

CONTACT_IDX = (8, 10, 20, 23, 31)
INSIDE_IDX = 5
DIST_SQ_THRESH = 4.0

_LBL_MASK_U = 0
for _ci in (INSIDE_IDX,) + CONTACT_IDX:
    _LBL_MASK_U |= 1 << _ci
_LBL_MASK = _LBL_MASK_U - (1 << 32) if _LBL_MASK_U >= (1 << 31) else _LBL_MASK_U

NC = 2
NS = 16
NW = NC * NS

_SC_PARAMS = pltpu.CompilerParams(
    needs_layout_passes=False, use_tc_tiling_on_sc=False)

_TW = 16


def _mesh():
    return plsc.VectorSubcoreMesh(core_axis_name="c", subcore_axis_name="s")


def _wid():
    return lax.axis_index("s") * NC + lax.axis_index("c")


def _full(c):
    return jnp.full((16,), c, jnp.int32)


_SW = 3136


def _make_stats(n):
    assert _SW * NW >= n and _SW % 16 == 0 and (n - _SW) % 8 == 0

    @functools.partial(
        pl.kernel,
        mesh=_mesh(),
        compiler_params=_SC_PARAMS,
        out_type=jax.ShapeDtypeStruct((n, _TW), jnp.float32),
        scratch_types=[
            pltpu.VMEM((24, _SW), jnp.float32),
            pltpu.VMEM((_SW, _TW), jnp.float32),
            pltpu.SemaphoreType.DMA,
        ],
    )
    def stats(ct_hbm, out_hbm, ct_v, sout_v, sem):
        base = jnp.minimum(_wid() * _SW, n - _SW)
        pltpu.async_copy(ct_hbm.at[:, pl.ds(base, _SW)], ct_v, sem).wait()
        iota16 = lax.iota(jnp.int32, 16)

        def group_body(i, carry):
            sl = pl.ds(i * 16, 16)
            rows = i * 16 + iota16
            for k in range(3):
                vs = [ct_v[k * 8 + c, sl] for c in range(8)]
                acc = vs[0]
                mn = vs[0]
                mx = vs[0]
                for v in vs[1:]:
                    acc = acc + v
                    mn = jnp.minimum(mn, v)
                    mx = jnp.maximum(mx, v)
                plsc.store_scatter(sout_v, [rows, _full(k)], acc * 0.125)
                plsc.store_scatter(sout_v, [rows, _full(3 + k)], mn)
                plsc.store_scatter(sout_v, [rows, _full(6 + k)], mx)
            return carry

        lax.fori_loop(0, _SW // 16, group_body, 0)
        pltpu.sync_copy(sout_v, out_hbm.at[pl.ds(base, _SW)])

    return stats


_CHUNK = 2000
_GB = 80
_NBMAX = _CHUNK // _GB
_WB = 5
_WAVE = _WB * _GB
_NWAVE = (_CHUNK + _WAVE - 1) // _WAVE
_SCAN_UNROLL = 5


def _make_veto(k_edges, n_nodes):
    pw = k_edges // NW
    nchunk = pw // _CHUNK
    assert pw % _CHUNK == 0 and pw % 8 == 0 and n_nodes % NS == 0
    assert nchunk >= 3

    @functools.partial(
        pl.kernel,
        mesh=_mesh(),
        compiler_params=_SC_PARAMS,
        out_type=jax.ShapeDtypeStruct((k_edges,), jnp.int32),
        scratch_types=[
            pltpu.VMEM((_CHUNK,), jnp.int32),
            pltpu.VMEM((_CHUNK,), jnp.int32),
            pltpu.VMEM((_CHUNK,), jnp.int32),
            pltpu.VMEM((_CHUNK,), jnp.int32),
            pltpu.VMEM((_CHUNK,), jnp.int32),
            pltpu.VMEM((_CHUNK,), jnp.int32),
            pltpu.VMEM((_CHUNK + 16,), jnp.int32),
            pltpu.VMEM((_NBMAX, _GB), jnp.int32),
            pltpu.VMEM((_NBMAX, _GB), jnp.int32),
            pltpu.VMEM((_WAVE, _TW), jnp.float32),
            pltpu.VMEM((_WAVE, _TW), jnp.float32),
            pltpu.VMEM_SHARED((n_nodes, _TW), jnp.float32),
            pltpu.SemaphoreType.DMA,
            pltpu.SemaphoreType.DMA,
            pltpu.SemaphoreType.DMA,
            pltpu.SemaphoreType.DMA,
        ],
    )
    def veto(stats_hbm, pidx_hbm, oidx_hbm, lbl_hbm, out_hbm,
             pidx_v, oidx_v, lbl_a, lbl_b, out_a, out_b, cidx_v,
             cpi_v, coi_v, prow_v, orow_v, stats_sh,
             sem_in, sem_lbl, sem_out, sem_g):
        base = _wid() * pw
        iota16 = lax.iota(jnp.int32, 16)
        ones16 = jnp.ones((16,), jnp.int32)
        zeros16 = jnp.zeros((16,), jnp.int32)
        bmask = jnp.full((16,), _LBL_MASK, jnp.int32)

        srows = n_nodes // NS
        sbase = lax.axis_index("s") * srows
        pltpu.async_copy(stats_hbm.at[pl.ds(sbase, srows)],
                         stats_sh.at[pl.ds(sbase, srows)], sem_in).wait()
        plsc.subcore_barrier()

        def init_body(i, carry):
            pos = i * 16 + iota16
            plsc.store_scatter(cpi_v, [pos // _GB, pos % _GB], zeros16)
            plsc.store_scatter(coi_v, [pos // _GB, pos % _GB], zeros16)
            return carry

        lax.fori_loop(0, _CHUNK // 16, init_body, 0)

        pltpu.async_copy(lbl_hbm.at[pl.ds(base, _CHUNK)], lbl_a, sem_lbl)

        def process(k, lbl_v, out_v, nxt_v, fire_next):
            cbase = base + k * _CHUNK
            cp_p = pltpu.async_copy(
                pidx_hbm.at[pl.ds(cbase, _CHUNK)], pidx_v, sem_in)
            cp_o = pltpu.async_copy(
                oidx_hbm.at[pl.ds(cbase, _CHUNK)], oidx_v, sem_in)
            pltpu.make_async_copy(
                lbl_hbm.at[pl.ds(base, _CHUNK)], lbl_v, sem_lbl).wait()
            if fire_next:
                pltpu.async_copy(
                    lbl_hbm.at[pl.ds(cbase + _CHUNK, _CHUNK)], nxt_v, sem_lbl)
            @pl.when(k >= 2)
            def _wait_out():
                pltpu.make_async_copy(
                    out_v, out_hbm.at[pl.ds(base, _CHUNK)], sem_out).wait()

            def scan_body(i, cnt):
                for u in range(_SCAN_UNROLL):
                    g = i * _SCAN_UNROLL + u
                    sl = pl.ds(g * 16, 16)
                    lbl = lbl_v[sl]
                    bit = lax.shift_right_logical(bmask, jnp.minimum(lbl, 31))
                    m = ((bit & 1) != 0) & (lbl <= 31)
                    out_v[sl] = ones16
                    plsc.store_compressed(
                        cidx_v.at[pl.ds(cnt, 16)], g * 16 + iota16, mask=m)
                    cnt = cnt + jnp.sum(m.astype(jnp.int32))
                return cnt

            cnt = lax.fori_loop(0, _CHUNK // (16 * _SCAN_UNROLL), scan_body, 0)
            ngrp = (cnt + 15) // 16
            cp_p.wait()
            cp_o.wait()

            def b1_body(g, carry2):
                sl = pl.ds(g * 16, 16)
                pos = g * 16 + iota16
                valid = pos < cnt
                eid = jnp.where(valid, cidx_v[sl], 0)
                plsc.store_scatter(cpi_v, [pos // _GB, pos % _GB],
                                   plsc.load_gather(pidx_v, [eid]))
                plsc.store_scatter(coi_v, [pos // _GB, pos % _GB],
                                   plsc.load_gather(oidx_v, [eid]))
                return carry2

            lax.fori_loop(0, ngrp, b1_body, 0)

            for w in range(_NWAVE):
                wb0 = w * _WAVE
                nb = min(_WB, _NBMAX - w * _WB)
                for b in range(nb):
                    @pl.when(wb0 + b * _GB < cnt)
                    def _fire(b=b, w=w):
                        sl = pl.ds(b * _GB, _GB)
                        pltpu.async_copy(
                            stats_sh.at[cpi_v.at[w * _WB + b]], prow_v.at[sl],
                            sem_g)
                        pltpu.async_copy(
                            stats_sh.at[coi_v.at[w * _WB + b]], orow_v.at[sl],
                            sem_g)
                for b in range(nb):
                    @pl.when(wb0 + b * _GB < cnt)
                    def _drain(b=b, w=w):
                        sl = pl.ds(b * _GB, _GB)
                        pltpu.make_async_copy(
                            stats_sh.at[cpi_v.at[w * _WB + b]], prow_v.at[sl],
                            sem_g).wait()
                        pltpu.make_async_copy(
                            stats_sh.at[coi_v.at[w * _WB + b]], orow_v.at[sl],
                            sem_g).wait()

                def b3_body(g, carry2, wb0=wb0):
                    rows = g * 16 + iota16
                    pos = wb0 + g * 16 + iota16
                    valid = pos < cnt
                    eid = jnp.where(
                        valid, cidx_v[pl.ds(wb0 + g * 16, 16)], 0)
                    lbl = plsc.load_gather(lbl_v, [eid])

                    def pcol(c):
                        return plsc.load_gather(prow_v, [rows, _full(c)])

                    def ocol(c):
                        return plsc.load_gather(orow_v, [rows, _full(c)])

                    ox, oy, oz = ocol(0), ocol(1), ocol(2)
                    dx = pcol(0) - ox
                    dy = pcol(1) - oy
                    dz = pcol(2) - oz
                    d2 = dx * dx + dy * dy + dz * dz
                    inb = ((ox >= pcol(3)) & (oy >= pcol(4)) & (oz >= pcol(5))
                           & (ox <= pcol(6)) & (oy <= pcol(7))
                           & (oz <= pcol(8)))
                    veto_m = jnp.where(lbl == INSIDE_IDX, ~inb,
                                       d2 > DIST_SQ_THRESH)
                    plsc.store_scatter(out_v, [eid], zeros16,
                                       mask=veto_m & valid)
                    return carry2

                ngrpw = jnp.clip(ngrp - wb0 // 16, 0, _WAVE // 16)
                lax.fori_loop(0, ngrpw, b3_body, 0)

            pltpu.async_copy(out_v, out_hbm.at[pl.ds(cbase, _CHUNK)], sem_out)

        def pair_body(k2, carry):
            process(2 * k2, lbl_a, out_a, lbl_b, True)
            process(2 * k2 + 1, lbl_b, out_b, lbl_a, True)
            return carry

        if nchunk % 2 == 0:
            lax.fori_loop(0, nchunk // 2 - 1, pair_body, 0)
            process(nchunk - 2, lbl_a, out_a, lbl_b, True)
            process(nchunk - 1, lbl_b, out_b, lbl_a, False)
            last_a, last_b = out_a, out_b
        else:
            lax.fori_loop(0, nchunk // 2, pair_body, 0)
            process(nchunk - 1, lbl_a, out_a, lbl_b, False)
            last_a, last_b = out_b, out_a
        pltpu.make_async_copy(
            last_a, out_hbm.at[pl.ds(base, _CHUNK)], sem_out).wait()
        pltpu.make_async_copy(
            last_b, out_hbm.at[pl.ds(base, _CHUNK)], sem_out).wait()

    return veto


def kernel(corners, person_idx, object_idx, pred_labels):
    n = corners.shape[0]
    k = person_idx.shape[0]
    ct = corners.transpose(2, 1, 0).reshape(24, n)
    stats = _make_stats(n)(ct)
    keep32 = _make_veto(k, n)(stats,
                              person_idx.astype(jnp.int32),
                              object_idx.astype(jnp.int32),
                              pred_labels.astype(jnp.int32))
    return keep32.astype(jnp.bool_)


# speedup vs baseline: 1.2783x; 1.0203x over previous
import functools

import jax
import jax.numpy as jnp
from jax import lax
from jax.experimental import pallas as pl
from jax.experimental.pallas import tpu as pltpu
from jax.experimental.pallas import tpu_sc as plsc

CONTACT_IDX = (8, 10, 20, 23, 31)
INSIDE_IDX = 5
DIST_SQ_THRESH = 4.0

_LBL_MASK_U = 0
for _ci in (INSIDE_IDX,) + CONTACT_IDX:
    _LBL_MASK_U |= 1 << _ci
_LBL_MASK = _LBL_MASK_U - (1 << 32) if _LBL_MASK_U >= (1 << 31) else _LBL_MASK_U

NC = 2
NS = 16
NW = NC * NS

_SC_PARAMS = pltpu.CompilerParams(
    needs_layout_passes=False, use_tc_tiling_on_sc=False)

_TW = 16


def _mesh():
    return plsc.VectorSubcoreMesh(core_axis_name="c", subcore_axis_name="s")


def _wid():
    return lax.axis_index("s") * NC + lax.axis_index("c")


def _full(c):
    return jnp.full((16,), c, jnp.int32)


_SW = 3136


def _make_stats(n):
    assert _SW * NW >= n and _SW % 16 == 0 and (n - _SW) % 8 == 0

    @functools.partial(
        pl.kernel,
        mesh=_mesh(),
        compiler_params=_SC_PARAMS,
        out_type=jax.ShapeDtypeStruct((n, _TW), jnp.float32),
        scratch_types=[
            pltpu.VMEM((24, _SW), jnp.float32),
            pltpu.VMEM((_SW, _TW), jnp.float32),
            pltpu.SemaphoreType.DMA,
        ],
    )
    def stats(ct_hbm, out_hbm, ct_v, sout_v, sem):
        base = jnp.minimum(_wid() * _SW, n - _SW)
        pltpu.async_copy(ct_hbm.at[:, pl.ds(base, _SW)], ct_v, sem).wait()
        iota16 = lax.iota(jnp.int32, 16)

        def group_body(i, carry):
            sl = pl.ds(i * 16, 16)
            rows = i * 16 + iota16
            for k in range(3):
                vs = [ct_v[k * 8 + c, sl] for c in range(8)]
                acc = vs[0]
                mn = vs[0]
                mx = vs[0]
                for v in vs[1:]:
                    acc = acc + v
                    mn = jnp.minimum(mn, v)
                    mx = jnp.maximum(mx, v)
                plsc.store_scatter(sout_v, [rows, _full(k)], acc * 0.125)
                plsc.store_scatter(sout_v, [rows, _full(3 + k)], mn)
                plsc.store_scatter(sout_v, [rows, _full(6 + k)], mx)
            return carry

        lax.fori_loop(0, _SW // 16, group_body, 0)
        pltpu.sync_copy(sout_v, out_hbm.at[pl.ds(base, _SW)])

    return stats


_CHUNK = 2000
_GB = 80
_NBMAX = _CHUNK // _GB
_WB = 5
_WAVE = _WB * _GB
_NWAVE = (_CHUNK + _WAVE - 1) // _WAVE
_SCAN_UNROLL = 1


def _make_veto(k_edges, n_nodes):
    pw = k_edges // NW
    nchunk = pw // _CHUNK
    assert pw % _CHUNK == 0 and pw % 8 == 0 and n_nodes % NS == 0
    assert nchunk >= 3

    @functools.partial(
        pl.kernel,
        mesh=_mesh(),
        compiler_params=_SC_PARAMS,
        out_type=jax.ShapeDtypeStruct((k_edges,), jnp.int32),
        scratch_types=[
            pltpu.VMEM((_CHUNK,), jnp.int32),
            pltpu.VMEM((_CHUNK,), jnp.int32),
            pltpu.VMEM((_CHUNK,), jnp.int32),
            pltpu.VMEM((_CHUNK,), jnp.int32),
            pltpu.VMEM((_CHUNK,), jnp.int32),
            pltpu.VMEM((_CHUNK,), jnp.int32),
            pltpu.VMEM((_CHUNK + 16,), jnp.int32),
            pltpu.VMEM((_NBMAX, _GB), jnp.int32),
            pltpu.VMEM((_NBMAX, _GB), jnp.int32),
            pltpu.VMEM((_WAVE, _TW), jnp.float32),
            pltpu.VMEM((_WAVE, _TW), jnp.float32),
            pltpu.VMEM_SHARED((n_nodes, _TW), jnp.float32),
            pltpu.SemaphoreType.DMA,
            pltpu.SemaphoreType.DMA,
            pltpu.SemaphoreType.DMA,
            pltpu.SemaphoreType.DMA,
        ],
    )
    def veto(stats_hbm, pidx_hbm, oidx_hbm, lbl_hbm, out_hbm,
             pidx_v, oidx_v, lbl_a, lbl_b, out_a, out_b, cidx_v,
             cpi_v, coi_v, prow_v, orow_v, stats_sh,
             sem_in, sem_lbl, sem_out, sem_g):
        base = _wid() * pw
        iota16 = lax.iota(jnp.int32, 16)
        ones16 = jnp.ones((16,), jnp.int32)
        zeros16 = jnp.zeros((16,), jnp.int32)
        bmask = jnp.full((16,), _LBL_MASK, jnp.int32)

        srows = n_nodes // NS
        sbase = lax.axis_index("s") * srows
        pltpu.async_copy(stats_hbm.at[pl.ds(sbase, srows)],
                         stats_sh.at[pl.ds(sbase, srows)], sem_in).wait()
        plsc.subcore_barrier()

        def init_body(i, carry):
            pos = i * 16 + iota16
            plsc.store_scatter(cpi_v, [pos // _GB, pos % _GB], zeros16)
            plsc.store_scatter(coi_v, [pos // _GB, pos % _GB], zeros16)
            return carry

        lax.fori_loop(0, _CHUNK // 16, init_body, 0)

        pltpu.async_copy(lbl_hbm.at[pl.ds(base, _CHUNK)], lbl_a, sem_lbl)

        def process(k, lbl_v, out_v, nxt_v, fire_next):
            cbase = base + k * _CHUNK
            cp_p = pltpu.async_copy(
                pidx_hbm.at[pl.ds(cbase, _CHUNK)], pidx_v, sem_in)
            cp_o = pltpu.async_copy(
                oidx_hbm.at[pl.ds(cbase, _CHUNK)], oidx_v, sem_in)
            pltpu.make_async_copy(
                lbl_hbm.at[pl.ds(base, _CHUNK)], lbl_v, sem_lbl).wait()
            if fire_next:
                pltpu.async_copy(
                    lbl_hbm.at[pl.ds(cbase + _CHUNK, _CHUNK)], nxt_v, sem_lbl)
            @pl.when(k >= 2)
            def _wait_out():
                pltpu.make_async_copy(
                    out_v, out_hbm.at[pl.ds(base, _CHUNK)], sem_out).wait()

            def scan_body(i, cnt):
                for u in range(_SCAN_UNROLL):
                    g = i * _SCAN_UNROLL + u
                    sl = pl.ds(g * 16, 16)
                    lbl = lbl_v[sl]
                    bit = lax.shift_right_logical(bmask, jnp.minimum(lbl, 31))
                    m = ((bit & 1) != 0) & (lbl <= 31)
                    out_v[sl] = ones16
                    plsc.store_compressed(
                        cidx_v.at[pl.ds(cnt, 16)], g * 16 + iota16, mask=m)
                    cnt = cnt + jnp.sum(m.astype(jnp.int32))
                return cnt

            cnt = lax.fori_loop(0, _CHUNK // (16 * _SCAN_UNROLL), scan_body, 0)
            ngrp = (cnt + 15) // 16
            cp_p.wait()
            cp_o.wait()

            def b1_body(g, carry2):
                sl = pl.ds(g * 16, 16)
                pos = g * 16 + iota16
                valid = pos < cnt
                eid = jnp.where(valid, cidx_v[sl], 0)
                plsc.store_scatter(cpi_v, [pos // _GB, pos % _GB],
                                   plsc.load_gather(pidx_v, [eid]))
                plsc.store_scatter(coi_v, [pos // _GB, pos % _GB],
                                   plsc.load_gather(oidx_v, [eid]))
                return carry2

            lax.fori_loop(0, ngrp, b1_body, 0)

            for w in range(_NWAVE):
                wb0 = w * _WAVE
                nb = min(_WB, _NBMAX - w * _WB)
                for b in range(nb):
                    @pl.when(wb0 + b * _GB < cnt)
                    def _fire(b=b, w=w):
                        sl = pl.ds(b * _GB, _GB)
                        pltpu.async_copy(
                            stats_sh.at[cpi_v.at[w * _WB + b]], prow_v.at[sl],
                            sem_g)
                        pltpu.async_copy(
                            stats_sh.at[coi_v.at[w * _WB + b]], orow_v.at[sl],
                            sem_g)
                for b in range(nb):
                    @pl.when(wb0 + b * _GB < cnt)
                    def _drain(b=b, w=w):
                        sl = pl.ds(b * _GB, _GB)
                        pltpu.make_async_copy(
                            stats_sh.at[cpi_v.at[w * _WB + b]], prow_v.at[sl],
                            sem_g).wait()
                        pltpu.make_async_copy(
                            stats_sh.at[coi_v.at[w * _WB + b]], orow_v.at[sl],
                            sem_g).wait()

                def b3_body(g, carry2, wb0=wb0):
                    rows = g * 16 + iota16
                    pos = wb0 + g * 16 + iota16
                    valid = pos < cnt
                    eid = jnp.where(
                        valid, cidx_v[pl.ds(wb0 + g * 16, 16)], 0)
                    lbl = plsc.load_gather(lbl_v, [eid])

                    def pcol(c):
                        return plsc.load_gather(prow_v, [rows, _full(c)])

                    def ocol(c):
                        return plsc.load_gather(orow_v, [rows, _full(c)])

                    ox, oy, oz = ocol(0), ocol(1), ocol(2)
                    dx = pcol(0) - ox
                    dy = pcol(1) - oy
                    dz = pcol(2) - oz
                    d2 = dx * dx + dy * dy + dz * dz
                    inb = ((ox >= pcol(3)) & (oy >= pcol(4)) & (oz >= pcol(5))
                           & (ox <= pcol(6)) & (oy <= pcol(7))
                           & (oz <= pcol(8)))
                    veto_m = jnp.where(lbl == INSIDE_IDX, ~inb,
                                       d2 > DIST_SQ_THRESH)
                    plsc.store_scatter(out_v, [eid], zeros16,
                                       mask=veto_m & valid)
                    return carry2

                ngrpw = jnp.clip(ngrp - wb0 // 16, 0, _WAVE // 16)
                lax.fori_loop(0, ngrpw, b3_body, 0)

            pltpu.async_copy(out_v, out_hbm.at[pl.ds(cbase, _CHUNK)], sem_out)

        def pair_body(k2, carry):
            process(2 * k2, lbl_a, out_a, lbl_b, True)
            process(2 * k2 + 1, lbl_b, out_b, lbl_a, True)
            return carry

        if nchunk % 2 == 0:
            lax.fori_loop(0, nchunk // 2 - 1, pair_body, 0)
            process(nchunk - 2, lbl_a, out_a, lbl_b, True)
            process(nchunk - 1, lbl_b, out_b, lbl_a, False)
            last_a, last_b = out_a, out_b
        else:
            lax.fori_loop(0, nchunk // 2, pair_body, 0)
            process(nchunk - 1, lbl_a, out_a, lbl_b, False)
            last_a, last_b = out_b, out_a
        pltpu.make_async_copy(
            last_a, out_hbm.at[pl.ds(base, _CHUNK)], sem_out).wait()
        pltpu.make_async_copy(
            last_b, out_hbm.at[pl.ds(base, _CHUNK)], sem_out).wait()

    return veto


def kernel(corners, person_idx, object_idx, pred_labels):
    n = corners.shape[0]
    k = person_idx.shape[0]
    ct = corners.transpose(2, 1, 0).reshape(24, n)
    stats = _make_stats(n)(ct)
    keep32 = _make_veto(k, n)(stats,
                              person_idx.astype(jnp.int32),
                              object_idx.astype(jnp.int32),
                              pred_labels.astype(jnp.int32))
    return keep32.astype(jnp.bool_)
